# baseline (speedup 1.0000x reference)
import functools

import jax
import jax.numpy as jnp
from jax import lax
from jax.experimental import pallas as pl
from jax.experimental.pallas import tpu as pltpu
from jax.experimental.pallas import tpu_sc as plsc

N_ROWS = 16384
DIM = 256
N_CODES = 8192
BLK_R = 256


ARGMIN_WINDOW = 2736


def _argmin_body(x_ref, emb_ref, xsq_ref, esq_ref, idx_ref):
    x_b = x_ref[...]
    e = emb_ref[...]
    c = lax.dot_general(x_b, e, (((1,), (1,)), ((), ())),
                        preferred_element_type=jnp.float32)
    dist = (xsq_ref[...] + esq_ref[...]) - 2.0 * c
    iota = lax.broadcasted_iota(jnp.int32, (BLK_R, N_CODES), 1)
    m = jnp.full((BLK_R, 1), jnp.inf, jnp.float32)
    jm = jnp.zeros((BLK_R, 1), jnp.int32)
    for lo in range(0, N_CODES, ARGMIN_WINDOW):
        hi = min(lo + ARGMIN_WINDOW, N_CODES)
        mask = (iota >= lo) & (iota < hi)
        dw = jnp.where(mask, dist, jnp.inf)
        v = jnp.min(dw, axis=1, keepdims=True)
        j = jnp.min(jnp.where(dw == v, iota, N_CODES), axis=1, keepdims=True)
        win = v < m
        jm = jnp.where(win, j, jm)
        m = jnp.where(win, v, m).astype(jnp.bfloat16).astype(jnp.float32)
    idx_ref[0, 0, :] = jm[:, 0]


def _tc_argmin(flat_x, emb, xsq, esq):
    nb = N_ROWS // BLK_R
    idx3 = pl.pallas_call(
        _argmin_body,
        grid=(nb,),
        in_specs=[
            pl.BlockSpec((BLK_R, DIM), lambda i: (i, 0)),
            pl.BlockSpec((N_CODES, DIM), lambda i: (0, 0)),
            pl.BlockSpec((BLK_R, 1), lambda i: (i, 0)),
            pl.BlockSpec((1, N_CODES), lambda i: (0, 0)),
        ],
        out_specs=pl.BlockSpec((1, 1, BLK_R), lambda i: (i, 0, 0)),
        out_shape=jax.ShapeDtypeStruct((nb, 1, BLK_R), jnp.int32),
        compiler_params=pltpu.CompilerParams(
            dimension_semantics=("parallel",)),
    )(flat_x, emb, xsq, esq)
    return idx3.reshape(N_ROWS)


def _make_sc_gather():
    info = plsc.get_sparse_core_info()
    nw = info.num_cores * info.num_subcores
    b_per_w = N_ROWS // nw
    chunk = 128
    n_chunks = b_per_w // chunk
    mesh = plsc.VectorSubcoreMesh(core_axis_name="c", subcore_axis_name="s")

    @functools.partial(
        pl.kernel,
        mesh=mesh,
        out_type=jax.ShapeDtypeStruct((N_ROWS, DIM), jnp.float32),
        scratch_types=[
            pltpu.VMEM((b_per_w,), jnp.int32),
            pltpu.VMEM((chunk, DIM), jnp.float32),
            pltpu.SemaphoreType.DMA,
        ],
    )
    def gather_k(idx_hbm, table_hbm, out_hbm, idx_v, rows_v, sem):
        wid = lax.axis_index("s") * info.num_cores + lax.axis_index("c")
        base = wid * b_per_w
        pltpu.sync_copy(idx_hbm.at[pl.ds(base, b_per_w)], idx_v)
        for j in range(n_chunks):
            pltpu.async_copy(
                table_hbm.at[idx_v.at[pl.ds(j * chunk, chunk)]], rows_v, sem
            ).wait()
            pltpu.sync_copy(rows_v, out_hbm.at[pl.ds(base + j * chunk, chunk)])

    return gather_k


def kernel(x, emb):
    flat_x = x.reshape(-1, DIM)
    xsq = jnp.sum(flat_x ** 2, axis=1, keepdims=True)
    esq = jnp.sum(emb ** 2, axis=1)[None, :]
    idx = _tc_argmin(flat_x, emb, xsq, esq)
    quant = _make_sc_gather()(idx, emb)
    return quant.reshape(x.shape), idx


# speedup vs baseline: 1.0748x; 1.0748x over previous
import functools

import jax
import jax.numpy as jnp
from jax import lax
from jax.experimental import pallas as pl
from jax.experimental.pallas import tpu as pltpu
from jax.experimental.pallas import tpu_sc as plsc

N_ROWS = 16384
DIM = 256
N_CODES = 8192
BLK_R = 256


ARGMIN_WINDOW = 2736


def _argmin_body(x_ref, emb_ref, xsq_ref, esq_ref, idx_ref):
    x_b = x_ref[...]
    e = emb_ref[...]
    c = lax.dot_general(x_b, e, (((1,), (1,)), ((), ())),
                        preferred_element_type=jnp.float32)
    dist = (xsq_ref[...] + esq_ref[...]) - 2.0 * c
    inf = jnp.float32(jnp.inf)
    lane = lax.broadcasted_iota(jnp.int32, (BLK_R, 128), 1)
    bt1 = dist[:, 2688:2816]
    bt2 = dist[:, 5376:5504]
    def _rmin(a):
        return jnp.min(a, axis=1, keepdims=True)
    v0 = jnp.minimum(_rmin(dist[:, :2688]),
                     _rmin(jnp.where(lane < 48, bt1, inf)))
    v1 = jnp.minimum(_rmin(jnp.where(lane >= 48, bt1, inf)),
                     jnp.minimum(_rmin(dist[:, 2816:5376]),
                                 _rmin(jnp.where(lane < 96, bt2, inf))))
    v2 = jnp.minimum(_rmin(jnp.where(lane >= 96, bt2, inf)),
                     _rmin(dist[:, 5504:]))
    m1 = v0.astype(jnp.bfloat16).astype(jnp.float32)
    win1 = v1 < m1
    m2 = jnp.where(win1, v1, m1).astype(jnp.bfloat16).astype(jnp.float32)
    win2 = v2 < m2
    vstar = jnp.where(win2, v2, jnp.where(win1, v1, v0))
    lo = jnp.where(win2, 5472, jnp.where(win1, 2736, 0))
    hi = jnp.where(win2, 8192, jnp.where(win1, 5472, 2736))
    iota_i = lax.broadcasted_iota(jnp.int32, (BLK_R, N_CODES), 1)
    iota_f = iota_i.astype(jnp.float32)
    hit = (iota_i >= lo) & (iota_i < hi) & (dist == vstar)
    idx_f = jnp.min(jnp.where(hit, iota_f, jnp.float32(N_CODES)), axis=1)
    idx_ref[0, 0, :] = idx_f.astype(jnp.int32)


def _tc_argmin(flat_x, emb, xsq, esq):
    nb = N_ROWS // BLK_R
    idx3 = pl.pallas_call(
        _argmin_body,
        grid=(nb,),
        in_specs=[
            pl.BlockSpec((BLK_R, DIM), lambda i: (i, 0)),
            pl.BlockSpec((N_CODES, DIM), lambda i: (0, 0)),
            pl.BlockSpec((BLK_R, 1), lambda i: (i, 0)),
            pl.BlockSpec((1, N_CODES), lambda i: (0, 0)),
        ],
        out_specs=pl.BlockSpec((1, 1, BLK_R), lambda i: (i, 0, 0)),
        out_shape=jax.ShapeDtypeStruct((nb, 1, BLK_R), jnp.int32),
        compiler_params=pltpu.CompilerParams(
            dimension_semantics=("parallel",)),
    )(flat_x, emb, xsq, esq)
    return idx3.reshape(N_ROWS)


def _make_sc_gather():
    info = plsc.get_sparse_core_info()
    nw = info.num_cores * info.num_subcores
    b_per_w = N_ROWS // nw
    chunk = 128
    n_chunks = b_per_w // chunk
    mesh = plsc.VectorSubcoreMesh(core_axis_name="c", subcore_axis_name="s")

    @functools.partial(
        pl.kernel,
        mesh=mesh,
        out_type=jax.ShapeDtypeStruct((N_ROWS, DIM), jnp.float32),
        scratch_types=[
            pltpu.VMEM((b_per_w,), jnp.int32),
            pltpu.VMEM((chunk, DIM), jnp.float32),
            pltpu.SemaphoreType.DMA,
        ],
    )
    def gather_k(idx_hbm, table_hbm, out_hbm, idx_v, rows_v, sem):
        wid = lax.axis_index("s") * info.num_cores + lax.axis_index("c")
        base = wid * b_per_w
        pltpu.sync_copy(idx_hbm.at[pl.ds(base, b_per_w)], idx_v)
        for j in range(n_chunks):
            pltpu.async_copy(
                table_hbm.at[idx_v.at[pl.ds(j * chunk, chunk)]], rows_v, sem
            ).wait()
            pltpu.sync_copy(rows_v, out_hbm.at[pl.ds(base + j * chunk, chunk)])

    return gather_k


def kernel(x, emb):
    flat_x = x.reshape(-1, DIM)
    xsq = jnp.sum(flat_x ** 2, axis=1, keepdims=True)
    esq = jnp.sum(emb ** 2, axis=1)[None, :]
    idx = _tc_argmin(flat_x, emb, xsq, esq)
    quant = _make_sc_gather()(idx, emb)
    return quant.reshape(x.shape), idx


# speedup vs baseline: 1.5663x; 1.4573x over previous
import functools

import jax
import jax.numpy as jnp
from jax import lax
from jax.experimental import pallas as pl
from jax.experimental.pallas import tpu as pltpu
from jax.experimental.pallas import tpu_sc as plsc

N_ROWS = 16384
DIM = 256
N_CODES = 8192
BLK_R = 256


ARGMIN_WINDOW = 2736


def _argmin_body(x_ref, emb_ref, xsq_ref, esq_ref, idx_ref):
    x_b = x_ref[...]
    e = emb_ref[...]
    c = lax.dot_general(x_b, e, (((1,), (1,)), ((), ())),
                        preferred_element_type=jnp.float32)
    dist = (xsq_ref[...] + esq_ref[...]) - 2.0 * c
    inf = jnp.float32(jnp.inf)
    lane = lax.broadcasted_iota(jnp.int32, (BLK_R, 128), 1)
    bt1 = dist[:, 2688:2816]
    bt2 = dist[:, 5376:5504]
    def _rmin(a):
        return jnp.min(a, axis=1, keepdims=True)
    v0 = jnp.minimum(_rmin(dist[:, :2688]),
                     _rmin(jnp.where(lane < 48, bt1, inf)))
    v1 = jnp.minimum(_rmin(jnp.where(lane >= 48, bt1, inf)),
                     jnp.minimum(_rmin(dist[:, 2816:5376]),
                                 _rmin(jnp.where(lane < 96, bt2, inf))))
    v2 = jnp.minimum(_rmin(jnp.where(lane >= 96, bt2, inf)),
                     _rmin(dist[:, 5504:]))
    m1 = v0.astype(jnp.bfloat16).astype(jnp.float32)
    win1 = v1 < m1
    m2 = jnp.where(win1, v1, m1).astype(jnp.bfloat16).astype(jnp.float32)
    win2 = v2 < m2
    vstar = jnp.where(win2, v2, jnp.where(win1, v1, v0))
    big = jnp.float32(N_CODES)
    iota_f = lax.broadcasted_iota(jnp.int32, (BLK_R, N_CODES), 1).astype(jnp.float32)
    def _scan(sl):
        a, b = sl.start or 0, sl.stop or N_CODES
        return jnp.min(jnp.where(dist[:, a:b] == vstar, iota_f[:, a:b], big),
                       axis=1, keepdims=True)
    def _scan_edge(sl, cond):
        a, b = sl.start, sl.stop
        return jnp.min(jnp.where((dist[:, a:b] == vstar) & cond,
                                 iota_f[:, a:b], big), axis=1, keepdims=True)
    j0 = jnp.minimum(_scan(slice(None, 2688)),
                     _scan_edge(slice(2688, 2816), lane < 48))
    j1 = jnp.minimum(_scan_edge(slice(2688, 2816), lane >= 48),
                     jnp.minimum(_scan(slice(2816, 5376)),
                                 _scan_edge(slice(5376, 5504), lane < 96)))
    j2 = jnp.minimum(_scan_edge(slice(5376, 5504), lane >= 96),
                     _scan(slice(5504, None)))
    idx_f = jnp.where(win2, j2, jnp.where(win1, j1, j0))
    idx_ref[0, 0, :] = idx_f[:, 0].astype(jnp.int32)


def _tc_argmin(flat_x, emb, xsq, esq):
    nb = N_ROWS // BLK_R
    idx3 = pl.pallas_call(
        _argmin_body,
        grid=(nb,),
        in_specs=[
            pl.BlockSpec((BLK_R, DIM), lambda i: (i, 0)),
            pl.BlockSpec((N_CODES, DIM), lambda i: (0, 0)),
            pl.BlockSpec((BLK_R, 1), lambda i: (i, 0)),
            pl.BlockSpec((1, N_CODES), lambda i: (0, 0)),
        ],
        out_specs=pl.BlockSpec((1, 1, BLK_R), lambda i: (i, 0, 0)),
        out_shape=jax.ShapeDtypeStruct((nb, 1, BLK_R), jnp.int32),
        compiler_params=pltpu.CompilerParams(
            dimension_semantics=("parallel",)),
    )(flat_x, emb, xsq, esq)
    return idx3.reshape(N_ROWS)


def _make_sc_gather():
    info = plsc.get_sparse_core_info()
    nw = info.num_cores * info.num_subcores
    b_per_w = N_ROWS // nw
    chunk = 128
    n_chunks = b_per_w // chunk
    mesh = plsc.VectorSubcoreMesh(core_axis_name="c", subcore_axis_name="s")

    @functools.partial(
        pl.kernel,
        mesh=mesh,
        out_type=jax.ShapeDtypeStruct((N_ROWS, DIM), jnp.float32),
        scratch_types=[
            pltpu.VMEM((b_per_w,), jnp.int32),
            pltpu.VMEM((chunk, DIM), jnp.float32),
            pltpu.SemaphoreType.DMA,
        ],
    )
    def gather_k(idx_hbm, table_hbm, out_hbm, idx_v, rows_v, sem):
        wid = lax.axis_index("s") * info.num_cores + lax.axis_index("c")
        base = wid * b_per_w
        pltpu.sync_copy(idx_hbm.at[pl.ds(base, b_per_w)], idx_v)
        for j in range(n_chunks):
            pltpu.async_copy(
                table_hbm.at[idx_v.at[pl.ds(j * chunk, chunk)]], rows_v, sem
            ).wait()
            pltpu.sync_copy(rows_v, out_hbm.at[pl.ds(base + j * chunk, chunk)])

    return gather_k


def kernel(x, emb):
    flat_x = x.reshape(-1, DIM)
    xsq = jnp.sum(flat_x ** 2, axis=1, keepdims=True)
    esq = jnp.sum(emb ** 2, axis=1)[None, :]
    idx = _tc_argmin(flat_x, emb, xsq, esq)
    quant = _make_sc_gather()(idx, emb)
    return quant.reshape(x.shape), idx


# speedup vs baseline: 1.6505x; 1.0538x over previous
import functools

import jax
import jax.numpy as jnp
from jax import lax
from jax.experimental import pallas as pl
from jax.experimental.pallas import tpu as pltpu
from jax.experimental.pallas import tpu_sc as plsc

N_ROWS = 16384
DIM = 256
N_CODES = 8192
BLK_R = 512


ARGMIN_WINDOW = 2736


def _argmin_body(x_ref, emb_ref, xsq_ref, esq_ref, idx_ref):
    x_b = x_ref[...]
    e = emb_ref[...]
    c = lax.dot_general(x_b, e, (((1,), (1,)), ((), ())),
                        preferred_element_type=jnp.float32)
    dist = (xsq_ref[...] + esq_ref[...]) - 2.0 * c
    inf = jnp.float32(jnp.inf)
    lane = lax.broadcasted_iota(jnp.int32, (BLK_R, 128), 1)
    bt1 = dist[:, 2688:2816]
    bt2 = dist[:, 5376:5504]
    def _rmin(a):
        return jnp.min(a, axis=1, keepdims=True)
    v0 = jnp.minimum(_rmin(dist[:, :2688]),
                     _rmin(jnp.where(lane < 48, bt1, inf)))
    v1 = jnp.minimum(_rmin(jnp.where(lane >= 48, bt1, inf)),
                     jnp.minimum(_rmin(dist[:, 2816:5376]),
                                 _rmin(jnp.where(lane < 96, bt2, inf))))
    v2 = jnp.minimum(_rmin(jnp.where(lane >= 96, bt2, inf)),
                     _rmin(dist[:, 5504:]))
    m1 = v0.astype(jnp.bfloat16).astype(jnp.float32)
    win1 = v1 < m1
    m2 = jnp.where(win1, v1, m1).astype(jnp.bfloat16).astype(jnp.float32)
    win2 = v2 < m2
    vstar = jnp.where(win2, v2, jnp.where(win1, v1, v0))
    big = jnp.float32(N_CODES)
    iota_f = lax.broadcasted_iota(jnp.int32, (BLK_R, N_CODES), 1).astype(jnp.float32)
    def _scan(sl):
        a, b = sl.start or 0, sl.stop or N_CODES
        return jnp.min(jnp.where(dist[:, a:b] == vstar, iota_f[:, a:b], big),
                       axis=1, keepdims=True)
    def _scan_edge(sl, cond):
        a, b = sl.start, sl.stop
        return jnp.min(jnp.where((dist[:, a:b] == vstar) & cond,
                                 iota_f[:, a:b], big), axis=1, keepdims=True)
    j0 = jnp.minimum(_scan(slice(None, 2688)),
                     _scan_edge(slice(2688, 2816), lane < 48))
    j1 = jnp.minimum(_scan_edge(slice(2688, 2816), lane >= 48),
                     jnp.minimum(_scan(slice(2816, 5376)),
                                 _scan_edge(slice(5376, 5504), lane < 96)))
    j2 = jnp.minimum(_scan_edge(slice(5376, 5504), lane >= 96),
                     _scan(slice(5504, None)))
    idx_f = jnp.where(win2, j2, jnp.where(win1, j1, j0))
    idx_ref[0, 0, :] = idx_f[:, 0].astype(jnp.int32)


def _tc_argmin(flat_x, emb, xsq, esq):
    nb = N_ROWS // BLK_R
    idx3 = pl.pallas_call(
        _argmin_body,
        grid=(nb,),
        in_specs=[
            pl.BlockSpec((BLK_R, DIM), lambda i: (i, 0)),
            pl.BlockSpec((N_CODES, DIM), lambda i: (0, 0)),
            pl.BlockSpec((BLK_R, 1), lambda i: (i, 0)),
            pl.BlockSpec((1, N_CODES), lambda i: (0, 0)),
        ],
        out_specs=pl.BlockSpec((1, 1, BLK_R), lambda i: (i, 0, 0)),
        out_shape=jax.ShapeDtypeStruct((nb, 1, BLK_R), jnp.int32),
        compiler_params=pltpu.CompilerParams(
            dimension_semantics=("parallel",)),
    )(flat_x, emb, xsq, esq)
    return idx3.reshape(N_ROWS)


def _make_sc_gather():
    info = plsc.get_sparse_core_info()
    nw = info.num_cores * info.num_subcores
    b_per_w = N_ROWS // nw
    chunk = 128
    n_chunks = b_per_w // chunk
    mesh = plsc.VectorSubcoreMesh(core_axis_name="c", subcore_axis_name="s")

    @functools.partial(
        pl.kernel,
        mesh=mesh,
        out_type=jax.ShapeDtypeStruct((N_ROWS, DIM), jnp.float32),
        scratch_types=[
            pltpu.VMEM((b_per_w,), jnp.int32),
            pltpu.VMEM((chunk, DIM), jnp.float32),
            pltpu.SemaphoreType.DMA,
        ],
    )
    def gather_k(idx_hbm, table_hbm, out_hbm, idx_v, rows_v, sem):
        wid = lax.axis_index("s") * info.num_cores + lax.axis_index("c")
        base = wid * b_per_w
        pltpu.sync_copy(idx_hbm.at[pl.ds(base, b_per_w)], idx_v)
        for j in range(n_chunks):
            pltpu.async_copy(
                table_hbm.at[idx_v.at[pl.ds(j * chunk, chunk)]], rows_v, sem
            ).wait()
            pltpu.sync_copy(rows_v, out_hbm.at[pl.ds(base + j * chunk, chunk)])

    return gather_k


def kernel(x, emb):
    flat_x = x.reshape(-1, DIM)
    xsq = jnp.sum(flat_x ** 2, axis=1, keepdims=True)
    esq = jnp.sum(emb ** 2, axis=1)[None, :]
    idx = _tc_argmin(flat_x, emb, xsq, esq)
    quant = _make_sc_gather()(idx, emb)
    return quant.reshape(x.shape), idx


# speedup vs baseline: 1.6542x; 1.0023x over previous
import functools

import jax
import jax.numpy as jnp
from jax import lax
from jax.experimental import pallas as pl
from jax.experimental.pallas import tpu as pltpu
from jax.experimental.pallas import tpu_sc as plsc

N_ROWS = 16384
DIM = 256
N_CODES = 8192
BLK_R = 512


def _argmin_body(x_ref, emb_ref, xsq_ref, esq_ref, idx_ref):
    x_b = x_ref[...]
    e = emb_ref[...]
    c = lax.dot_general(x_b, e, (((1,), (1,)), ((), ())),
                        preferred_element_type=jnp.float32)
    dist = (xsq_ref[...] + esq_ref[...]) - 2.0 * c
    inf = jnp.float32(jnp.inf)
    lane = lax.broadcasted_iota(jnp.int32, (BLK_R, 128), 1)
    bt1 = dist[:, 2688:2816]
    bt2 = dist[:, 5376:5504]
    def _rmin(a):
        return jnp.min(a, axis=1, keepdims=True)
    v0 = jnp.minimum(_rmin(dist[:, :2688]),
                     _rmin(jnp.where(lane < 48, bt1, inf)))
    v1 = jnp.minimum(_rmin(jnp.where(lane >= 48, bt1, inf)),
                     jnp.minimum(_rmin(dist[:, 2816:5376]),
                                 _rmin(jnp.where(lane < 96, bt2, inf))))
    v2 = jnp.minimum(_rmin(jnp.where(lane >= 96, bt2, inf)),
                     _rmin(dist[:, 5504:]))
    m1 = v0.astype(jnp.bfloat16).astype(jnp.float32)
    win1 = v1 < m1
    m2 = jnp.where(win1, v1, m1).astype(jnp.bfloat16).astype(jnp.float32)
    win2 = v2 < m2
    vstar = jnp.where(win2, v2, jnp.where(win1, v1, v0))
    big = jnp.float32(N_CODES)
    iota_f = lax.broadcasted_iota(jnp.int32, (BLK_R, N_CODES), 1).astype(jnp.float32)
    def _scan(sl):
        a, b = sl.start or 0, sl.stop or N_CODES
        return jnp.min(jnp.where(dist[:, a:b] == vstar, iota_f[:, a:b], big),
                       axis=1, keepdims=True)
    def _scan_edge(sl, cond):
        a, b = sl.start, sl.stop
        return jnp.min(jnp.where((dist[:, a:b] == vstar) & cond,
                                 iota_f[:, a:b], big), axis=1, keepdims=True)
    j0 = jnp.minimum(_scan(slice(None, 2688)),
                     _scan_edge(slice(2688, 2816), lane < 48))
    j1 = jnp.minimum(_scan_edge(slice(2688, 2816), lane >= 48),
                     jnp.minimum(_scan(slice(2816, 5376)),
                                 _scan_edge(slice(5376, 5504), lane < 96)))
    j2 = jnp.minimum(_scan_edge(slice(5376, 5504), lane >= 96),
                     _scan(slice(5504, None)))
    idx_f = jnp.where(win2, j2, jnp.where(win1, j1, j0))
    idx_ref[0, 0, :] = idx_f[:, 0].astype(jnp.int32)


def _tc_argmin(flat_x, emb, xsq, esq):
    nb = N_ROWS // BLK_R
    idx3 = pl.pallas_call(
        _argmin_body,
        grid=(nb,),
        in_specs=[
            pl.BlockSpec((BLK_R, DIM), lambda i: (i, 0)),
            pl.BlockSpec((N_CODES, DIM), lambda i: (0, 0)),
            pl.BlockSpec((BLK_R, 1), lambda i: (i, 0)),
            pl.BlockSpec((1, N_CODES), lambda i: (0, 0)),
        ],
        out_specs=pl.BlockSpec((1, 1, BLK_R), lambda i: (i, 0, 0)),
        out_shape=jax.ShapeDtypeStruct((nb, 1, BLK_R), jnp.int32),
    )(flat_x, emb, xsq, esq)
    return idx3.reshape(N_ROWS)


def _make_sc_gather():
    info = plsc.get_sparse_core_info()
    nw = info.num_cores * info.num_subcores
    b_per_w = N_ROWS // nw
    chunk = 128
    n_chunks = b_per_w // chunk
    mesh = plsc.VectorSubcoreMesh(core_axis_name="c", subcore_axis_name="s")

    @functools.partial(
        pl.kernel,
        mesh=mesh,
        out_type=jax.ShapeDtypeStruct((N_ROWS, DIM), jnp.float32),
        scratch_types=[
            pltpu.VMEM((b_per_w,), jnp.int32),
            pltpu.VMEM((chunk, DIM), jnp.float32),
            pltpu.SemaphoreType.DMA,
        ],
    )
    def gather_k(idx_hbm, table_hbm, out_hbm, idx_v, rows_v, sem):
        wid = lax.axis_index("s") * info.num_cores + lax.axis_index("c")
        base = wid * b_per_w
        pltpu.sync_copy(idx_hbm.at[pl.ds(base, b_per_w)], idx_v)
        for j in range(n_chunks):
            pltpu.async_copy(
                table_hbm.at[idx_v.at[pl.ds(j * chunk, chunk)]], rows_v, sem
            ).wait()
            pltpu.sync_copy(rows_v, out_hbm.at[pl.ds(base + j * chunk, chunk)])

    return gather_k


def kernel(x, emb):
    flat_x = x.reshape(-1, DIM)
    xsq = jnp.sum(flat_x ** 2, axis=1, keepdims=True)
    esq = jnp.sum(emb ** 2, axis=1)[None, :]
    idx = _tc_argmin(flat_x, emb, xsq, esq)
    quant = _make_sc_gather()(idx, emb)
    return quant.reshape(x.shape), idx
